# baseline (speedup 1.0000x reference)
import functools

import jax
import jax.numpy as jnp
from jax import lax
from jax.experimental import pallas as pl
from jax.experimental.pallas import tpu as pltpu
from jax.experimental.pallas import tpu_sc as plsc

NC = 2
NS = 16
NW = NC * NS
G = 128
NBUF = 3


def _sc_join(unary, binary, idx1g, idx2g, NG):
    V, D = unary.shape
    B, E = binary.shape
    W = 2 * D + E
    base_pw = NG // NW
    rem = NG - base_pw * NW
    stage = ((base_pw + 7) // 8) * 8 + 8
    assert base_pw % NBUF == 0 and base_pw >= 2 * NBUF
    assert idx1g.shape[0] >= ((NW - 1) * base_pw // 8) * 8 + stage

    mesh = plsc.VectorSubcoreMesh(core_axis_name="c", subcore_axis_name="s")

    @functools.partial(
        pl.kernel,
        out_type=jax.ShapeDtypeStruct((B, W), jnp.float32),
        mesh=mesh,
        scratch_types=[
            pltpu.VMEM((stage, G), jnp.int32),
            pltpu.VMEM((stage, G), jnp.int32),
            pltpu.VMEM((NBUF, G, D), jnp.float32),
            pltpu.VMEM((NBUF, G, D), jnp.float32),
            [pltpu.SemaphoreType.DMA] * NBUF,
            [pltpu.SemaphoreType.DMA] * NBUF,
        ],
    )
    def join_kernel(unary_h, binary_h, idx1_h, idx2_h, out_h,
                    i1_v, i2_v, r1_v, r2_v, sem_in, sem_out):
        cid = lax.axis_index("c")
        sid = lax.axis_index("s")
        wid = sid * NC + cid
        g0 = wid * base_pw

        g0_al = pl.multiple_of((g0 // 8) * 8, 8)
        off = g0 - g0_al
        pltpu.sync_copy(idx1_h.at[pl.ds(g0_al, stage)], i1_v)
        pltpu.sync_copy(idx2_h.at[pl.ds(g0_al, stage)], i2_v)

        def issue_in(g, b):
            pltpu.async_copy(unary_h.at[i1_v.at[off + g]], r1_v.at[b], sem_in[b])
            pltpu.async_copy(unary_h.at[i2_v.at[off + g]], r2_v.at[b], sem_in[b])

        def wait_in(b):
            pltpu.make_async_copy(unary_h.at[pl.ds(0, G)], r1_v.at[b],
                                  sem_in[b]).wait()
            pltpu.make_async_copy(unary_h.at[pl.ds(0, G)], r2_v.at[b],
                                  sem_in[b]).wait()

        def issue_out(g, b):
            row = (g0 + g) * G
            pltpu.async_copy(r1_v.at[b], out_h.at[pl.ds(row, G), pl.ds(0, D)],
                             sem_out[b])
            pltpu.async_copy(r2_v.at[b], out_h.at[pl.ds(row, G), pl.ds(D, D)],
                             sem_out[b])
            pltpu.async_copy(binary_h.at[pl.ds(row, G)],
                             out_h.at[pl.ds(row, G), pl.ds(2 * D, E)],
                             sem_out[b])

        def wait_out(b):
            pltpu.make_async_copy(r1_v.at[b], out_h.at[pl.ds(0, G), pl.ds(0, D)],
                                  sem_out[b]).wait()
            pltpu.make_async_copy(r2_v.at[b], out_h.at[pl.ds(0, G), pl.ds(D, D)],
                                  sem_out[b]).wait()
            pltpu.make_async_copy(binary_h.at[pl.ds(0, G)],
                                  out_h.at[pl.ds(0, G), pl.ds(2 * D, E)],
                                  sem_out[b]).wait()

        def slot(g, b, do_waitout, do_issue):
            wait_in(b)
            issue_out(g, b)
            bm = (b + 2) % NBUF
            if do_waitout:
                wait_out(bm)
            if do_issue:
                issue_in(g + 2, bm)

        issue_in(0, 0)
        issue_in(1, 1)

        slot(0, 0, False, True)
        slot(1, 1, True, True)
        slot(2, 2, True, True)

        @pl.loop(NBUF, base_pw - NBUF, step=NBUF)
        def _(jj):
            for b in range(NBUF):
                slot(jj + b, b, True, True)

        slot(base_pw - 3, 0, True, True)
        slot(base_pw - 2, 1, True, False)
        slot(base_pw - 1, 2, True, False)
        wait_out(2)

        @pl.when(wid < rem)
        def _():
            eg = NW * base_pw + wid
            row = eg * G
            pltpu.sync_copy(idx1_h.at[eg], i1_v.at[0])
            pltpu.sync_copy(idx2_h.at[eg], i2_v.at[0])
            c1 = pltpu.async_copy(unary_h.at[i1_v.at[0]], r1_v.at[0], sem_in[0])
            c2 = pltpu.async_copy(unary_h.at[i2_v.at[0]], r2_v.at[0], sem_in[0])
            pltpu.sync_copy(binary_h.at[pl.ds(row, G)],
                            out_h.at[pl.ds(row, G), pl.ds(2 * D, E)])
            c1.wait()
            c2.wait()
            pltpu.sync_copy(r1_v.at[0], out_h.at[pl.ds(row, G), pl.ds(0, D)])
            pltpu.sync_copy(r2_v.at[0], out_h.at[pl.ds(row, G), pl.ds(D, D)])

    return join_kernel(unary, binary, idx1g, idx2g)


def kernel(unary, binary, index1, index2):
    B = index1.shape[0]
    NG = B // G
    NG_pad = ((NG + 7) // 8) * 8 + 16
    idx1g = jnp.zeros((NG_pad, G), index1.dtype).at[:NG].set(
        index1.reshape(NG, G))
    idx2g = jnp.zeros((NG_pad, G), index2.dtype).at[:NG].set(
        index2.reshape(NG, G))
    return _sc_join(unary, binary, idx1g, idx2g, NG)


# speedup vs baseline: 6.6047x; 6.6047x over previous
import functools

import jax
import jax.numpy as jnp
from jax import lax
from jax.experimental import pallas as pl
from jax.experimental.pallas import tpu as pltpu
from jax.experimental.pallas import tpu_sc as plsc

NC = 2
NS = 16
NW = NC * NS
G = 128
NBUF = 2


def _sc_join(unary, binary, idx1g, idx2g, NG):
    V, D = unary.shape
    B, E = binary.shape
    W = 2 * D + E
    base_pw = NG // NW
    rem = NG - base_pw * NW
    stage = ((base_pw + 7) // 8) * 8 + 8
    assert base_pw % 2 == 0 and base_pw >= 4
    assert idx1g.shape[0] >= ((NW - 1) * base_pw // 8) * 8 + stage

    mesh = plsc.VectorSubcoreMesh(core_axis_name="c", subcore_axis_name="s")

    @functools.partial(
        pl.kernel,
        out_type=jax.ShapeDtypeStruct((B, W), jnp.float32),
        mesh=mesh,
        scratch_types=[
            pltpu.VMEM((stage, G), jnp.int32),
            pltpu.VMEM((stage, G), jnp.int32),
            pltpu.VMEM((NBUF, G, D), jnp.float32),
            pltpu.VMEM((NBUF, G, D), jnp.float32),
            pltpu.VMEM((NBUF, G, E), jnp.float32),
            [pltpu.SemaphoreType.DMA] * NBUF,
            [pltpu.SemaphoreType.DMA] * NBUF,
        ],
    )
    def join_kernel(unary_h, binary_h, idx1_h, idx2_h, out_h,
                    i1_v, i2_v, r1_v, r2_v, b_v, sem_in, sem_out):
        cid = lax.axis_index("c")
        sid = lax.axis_index("s")
        wid = sid * NC + cid
        g0 = wid * base_pw

        g0_al = pl.multiple_of((g0 // 8) * 8, 8)
        off = g0 - g0_al
        pltpu.sync_copy(idx1_h.at[pl.ds(g0_al, stage)], i1_v)
        pltpu.sync_copy(idx2_h.at[pl.ds(g0_al, stage)], i2_v)

        def issue_in(g, b):
            pltpu.async_copy(unary_h.at[i1_v.at[off + g]], r1_v.at[b], sem_in[b])
            pltpu.async_copy(unary_h.at[i2_v.at[off + g]], r2_v.at[b], sem_in[b])
            pltpu.async_copy(binary_h.at[pl.ds((g0 + g) * G, G)], b_v.at[b],
                             sem_in[b])

        def wait_in(b):
            pltpu.make_async_copy(unary_h.at[pl.ds(0, G)], r1_v.at[b],
                                  sem_in[b]).wait()
            pltpu.make_async_copy(unary_h.at[pl.ds(0, G)], r2_v.at[b],
                                  sem_in[b]).wait()
            pltpu.make_async_copy(binary_h.at[pl.ds(0, G)], b_v.at[b],
                                  sem_in[b]).wait()

        def issue_out(g, b):
            row = (g0 + g) * G
            pltpu.async_copy(r1_v.at[b], out_h.at[pl.ds(row, G), pl.ds(0, D)],
                             sem_out[b])
            pltpu.async_copy(r2_v.at[b], out_h.at[pl.ds(row, G), pl.ds(D, D)],
                             sem_out[b])
            pltpu.async_copy(b_v.at[b], out_h.at[pl.ds(row, G), pl.ds(2 * D, E)],
                             sem_out[b])

        def wait_out(b):
            pltpu.make_async_copy(r1_v.at[b], out_h.at[pl.ds(0, G), pl.ds(0, D)],
                                  sem_out[b]).wait()
            pltpu.make_async_copy(r2_v.at[b], out_h.at[pl.ds(0, G), pl.ds(D, D)],
                                  sem_out[b]).wait()
            pltpu.make_async_copy(b_v.at[b], out_h.at[pl.ds(0, G),
                                                      pl.ds(2 * D, E)],
                                  sem_out[b]).wait()

        def slot(g, b, do_waitout, do_issue):
            wait_in(b)
            issue_out(g, b)
            bm = 1 - b
            if do_waitout:
                wait_out(bm)
            if do_issue:
                issue_in(g + 1, bm)

        issue_in(0, 0)
        slot(0, 0, False, True)

        @pl.loop(1, base_pw - 1, step=2)
        def _(jj):
            slot(jj, 1, True, True)
            slot(jj + 1, 0, True, True)

        slot(base_pw - 1, 1, True, False)
        wait_out(1)

        @pl.when(wid < rem)
        def _():
            eg = NW * base_pw + wid
            row = eg * G
            pltpu.sync_copy(idx1_h.at[eg], i1_v.at[0])
            pltpu.sync_copy(idx2_h.at[eg], i2_v.at[0])
            c1 = pltpu.async_copy(unary_h.at[i1_v.at[0]], r1_v.at[0], sem_in[0])
            c2 = pltpu.async_copy(unary_h.at[i2_v.at[0]], r2_v.at[0], sem_in[0])
            pltpu.sync_copy(binary_h.at[pl.ds(row, G)], b_v.at[0])
            c1.wait()
            c2.wait()
            pltpu.sync_copy(r1_v.at[0], out_h.at[pl.ds(row, G), pl.ds(0, D)])
            pltpu.sync_copy(r2_v.at[0], out_h.at[pl.ds(row, G), pl.ds(D, D)])
            pltpu.sync_copy(b_v.at[0], out_h.at[pl.ds(row, G), pl.ds(2 * D, E)])

    return join_kernel(unary, binary, idx1g, idx2g)


def kernel(unary, binary, index1, index2):
    B = index1.shape[0]
    NG = B // G
    NG_pad = ((NG + 7) // 8) * 8 + 16
    idx1g = jnp.zeros((NG_pad, G), index1.dtype).at[:NG].set(
        index1.reshape(NG, G))
    idx2g = jnp.zeros((NG_pad, G), index2.dtype).at[:NG].set(
        index2.reshape(NG, G))
    return _sc_join(unary, binary, idx1g, idx2g, NG)
